# baseline (speedup 1.0000x reference)
import functools
import math

import jax
import jax.numpy as jnp
from jax import lax
from jax.experimental import pallas as pl
from jax.experimental.pallas import tpu as pltpu
from jax.experimental.pallas import tpu_sc as plsc

N = 10000
E = 320000
D = 128
NSPEC = 4
NB = 8
HID = 8

BE = 6400
BN = 2000
EPB = 16


def _edge_w_body(emb_ref, ai_ref, rep_ref, wfc1bd_ref, wfc2e_ref, out_ref):
    hi = jax.lax.Precision.HIGHEST
    h = jnp.dot(emb_ref[...], wfc1bd_ref[...], precision=hi,
                preferred_element_type=jnp.float32) * (1.0 / math.sqrt(NB))
    h = h * jax.nn.sigmoid(h)
    a_int = jnp.dot(ai_ref[...], rep_ref[...], preferred_element_type=jnp.float32)
    h = h * a_int * (1.0 / math.sqrt(HID))
    wfc2e = wfc2e_ref[...]
    parts = [jnp.dot(h, wfc2e[t], precision=hi,
                     preferred_element_type=jnp.float32)
             for t in range(EPB)]
    out_ref[...] = jnp.concatenate(parts, axis=0)


def _edge_w(emb2, ai16, rep, wfc1bd, wfc2e):
    R = BE // EPB
    return pl.pallas_call(
        _edge_w_body,
        grid=(E // BE,),
        in_specs=[
            pl.BlockSpec((R, D), lambda i: (i, 0)),
            pl.BlockSpec((R, EPB), lambda i: (i, 0)),
            pl.BlockSpec((EPB, D), lambda i: (0, 0)),
            pl.BlockSpec((D, D), lambda i: (0, 0)),
            pl.BlockSpec((EPB, D, D), lambda i: (0, 0, 0)),
        ],
        out_specs=pl.BlockSpec((BE, D), lambda i: (i, 0)),
        out_shape=jax.ShapeDtypeStruct((E, D), jnp.float32),
    )(emb2, ai16, rep, wfc1bd, wfc2e)


def _node_pre_body(nf_ref, attrs_ref, w1_ref, wsct_ref, xw_ref, sc_ref):
    nf = nf_ref[...]
    xw_ref[...] = jnp.dot(nf, w1_ref[...],
                          preferred_element_type=jnp.float32) * (1.0 / math.sqrt(D))
    wsct = wsct_ref[...]
    attrs = attrs_ref[...]
    acc = attrs[:, 0:1] * jnp.dot(nf, wsct[0], preferred_element_type=jnp.float32)
    for j in range(1, NSPEC):
        acc = acc + attrs[:, j:j + 1] * jnp.dot(
            nf, wsct[j], preferred_element_type=jnp.float32)
    sc_ref[...] = acc * (1.0 / math.sqrt(D * NSPEC))


def _node_pre(nf, attrs, w1, wsct):
    return pl.pallas_call(
        _node_pre_body,
        grid=(N // BN,),
        in_specs=[
            pl.BlockSpec((BN, D), lambda i: (i, 0)),
            pl.BlockSpec((BN, NSPEC), lambda i: (i, 0)),
            pl.BlockSpec((D, D), lambda i: (0, 0)),
            pl.BlockSpec((NSPEC, D, D), lambda i: (0, 0, 0)),
        ],
        out_specs=[
            pl.BlockSpec((BN, D), lambda i: (i, 0)),
            pl.BlockSpec((BN, D), lambda i: (i, 0)),
        ],
        out_shape=[
            jax.ShapeDtypeStruct((N, D), jnp.float32),
            jax.ShapeDtypeStruct((N, D), jnp.float32),
        ],
    )(nf, attrs, w1, wsct)


NC = 2
NS = 16
NW = NC * NS
EPT = E // NW
C = 80
NCH = EPT // C
NPT = 624


RPB = BE // EPB
WL = C * EPB


def _sc_body(xw_hbm, w_hbm, src_hbm, dst_hbm, out_hbm,
             srcw0, srcw1, dstw0, dstw1, sidx0, sidx1, didx0, didx1,
             rows0, rows1, wbuf0, wbuf1,
             acc, ssw0, ssw1, sdw0, sdw1, sw0, sw1, sg0, sg1):
    c = lax.axis_index("c")
    s = lax.axis_index("s")
    wid = c * NS + s
    ebase = wid * EPT
    nbase = s * NPT
    srcw = (srcw0, srcw1)
    dstw = (dstw0, dstw1)
    sidx = (sidx0, sidx1)
    didx = (didx0, didx1)
    rows = (rows0, rows1)
    wbuf = (wbuf0, wbuf1)
    ssw = (ssw0, ssw1)
    sdw = (sdw0, sdw1)
    sw = (sw0, sw1)
    sg = (sg0, sg1)

    def _zero(i, _):
        for j in range(D // 16):
            rows0[i, pl.ds(j * 16, 16)] = jnp.zeros((16,), jnp.float32)
        return 0

    lax.fori_loop(0, C, _zero, 0)
    for k in range(NPT // C):
        pltpu.sync_copy(rows0, acc.at[pl.ds(nbase + k * C, C)])
    rem = NPT % C
    if rem:
        pltpu.sync_copy(rows0.at[pl.ds(0, rem)],
                        acc.at[pl.ds(nbase + (NPT // C) * C, rem)])

    @pl.when(s == NS - 1)
    def _zero_tail():
        pltpu.sync_copy(rows0.at[pl.ds(0, N - NS * NPT)],
                        acc.at[pl.ds(NS * NPT, N - NS * NPT)])

    plsc.subcore_barrier()

    def _winparams(i):
        P = ebase + i * C
        g = P // BE
        q = P % BE
        t = q // RPB
        W = g * BE + EPB * (q % RPB)
        return P, W, t

    def _idx_w(i, b):
        P, W, _ = _winparams(i)
        pltpu.async_copy(src_hbm.at[pl.ds(W, WL)], srcw[b], ssw[b])
        pltpu.async_copy(dst_hbm.at[pl.ds(W, WL)], dstw[b], sdw[b])
        pltpu.async_copy(w_hbm.at[pl.ds(P, C)], wbuf[b], sw[b])

    def _extract_gather(i, b):
        _, _, t = _winparams(i)
        pltpu.make_async_copy(src_hbm.at[pl.ds(0, WL)], srcw[b], ssw[b]).wait()
        pltpu.make_async_copy(dst_hbm.at[pl.ds(0, WL)], dstw[b], sdw[b]).wait()
        iota = lax.broadcasted_iota(jnp.int32, (16,), 0) * EPB
        for c5 in range(C // 16):
            idx = iota + (16 * EPB * c5 + t)
            sidx[b][pl.ds(c5 * 16, 16)] = plsc.load_gather(srcw[b], [idx])
            didx[b][pl.ds(c5 * 16, 16)] = plsc.load_gather(dstw[b], [idx])
        pltpu.async_copy(xw_hbm.at[sidx[b]], rows[b], sg[b])

    def _step(i, b, prefetch):
        if prefetch:
            _idx_w(i + 1, b ^ 1)
        pltpu.make_async_copy(w_hbm.at[pl.ds(0, C)], wbuf[b], sw[b]).wait()
        pltpu.make_async_copy(xw_hbm.at[sidx[b]], rows[b], sg[b]).wait()

        def _mul(e, _):
            for j in range(D // 16):
                sl = pl.ds(j * 16, 16)
                rows[b][e, sl] = rows[b][e, sl] * wbuf[b][e, sl]
            return 0

        lax.fori_loop(0, C, _mul, 0)
        if prefetch:
            _extract_gather(i + 1, b ^ 1)
        pltpu.sync_copy(rows[b], acc.at[didx[b]], add=True)

    _idx_w(0, 0)
    _extract_gather(0, 0)

    def _pair(i2, _):
        i = i2 * 2
        _step(i, 0, True)
        _step(i + 1, 1, True)
        return 0

    lax.fori_loop(0, (NCH - 1) // 2, _pair, 0)
    _step(NCH - 1, (NCH - 1) & 1, False)

    plsc.subcore_barrier()
    pltpu.sync_copy(acc.at[pl.ds(nbase, NPT)],
                    out_hbm.at[c, pl.ds(nbase, NPT)])

    @pl.when(s == NS - 1)
    def _out_tail():
        pltpu.sync_copy(acc.at[pl.ds(NS * NPT, N - NS * NPT)],
                        out_hbm.at[c, pl.ds(NS * NPT, N - NS * NPT)])


def _sc_scatter(xw, w, src, dst):
    mesh = plsc.VectorSubcoreMesh(core_axis_name="c", subcore_axis_name="s")
    f = pl.kernel(
        _sc_body,
        out_type=jax.ShapeDtypeStruct((NC, N, D), jnp.float32),
        mesh=mesh,
        compiler_params=pltpu.CompilerParams(needs_layout_passes=False),
        scratch_types=(
            [pltpu.VMEM((WL,), jnp.int32)] * 4
            + [pltpu.VMEM((C,), jnp.int32)] * 4
            + [pltpu.VMEM((C, D), jnp.float32)] * 4
            + [pltpu.VMEM_SHARED((N, D), jnp.float32)]
            + [pltpu.SemaphoreType.DMA] * 8
        ),
    )
    return f(xw, w, src, dst)


def _final_body(p_ref, sc_ref, w2_ref, out_ref):
    ssum = p_ref[0] + p_ref[1]
    out_ref[...] = jnp.dot(ssum, w2_ref[...],
                           preferred_element_type=jnp.float32) * (1.0 / math.sqrt(D)) + sc_ref[...]


def _final(partials, scv, w2):
    return pl.pallas_call(
        _final_body,
        grid=(N // BN,),
        in_specs=[
            pl.BlockSpec((NC, BN, D), lambda i: (0, i, 0)),
            pl.BlockSpec((BN, D), lambda i: (i, 0)),
            pl.BlockSpec((D, D), lambda i: (0, 0)),
        ],
        out_specs=pl.BlockSpec((BN, D), lambda i: (i, 0)),
        out_shape=jax.ShapeDtypeStruct((N, D), jnp.float32),
    )(partials, scv, w2)


import numpy as np

_REP = np.zeros((EPB, D), np.float32)
_REP[np.arange(EPB)[:, None], np.arange(EPB * NB).reshape(EPB, NB)] = 1.0


def kernel(node_features, node_attrs, edge_attrs, edge_embedding,
           W1, Wfc1, Wfc2, W2, Wsc, edge_index):
    src = edge_index[1]
    dst = edge_index[0]
    wsct = jnp.transpose(Wsc, (1, 0, 2))
    emb2 = edge_embedding.reshape(E // EPB, D)
    ai16 = edge_attrs.reshape(E // EPB, EPB)
    rep = jnp.asarray(_REP)
    wfc1bd = jnp.kron(jnp.eye(EPB, dtype=jnp.float32), Wfc1)
    wfc2e = jnp.zeros((EPB, D, D), jnp.float32).at[
        np.arange(EPB)[:, None], np.arange(EPB * NB).reshape(EPB, NB)].set(Wfc2)
    w = _edge_w(emb2, ai16, rep, wfc1bd, wfc2e)
    xw, scv = _node_pre(node_features, node_attrs, W1, wsct)
    partials = _sc_scatter(xw, w, src, dst)
    return _final(partials, scv, W2)


# speedup vs baseline: 1.1696x; 1.1696x over previous
import functools
import math

import jax
import jax.numpy as jnp
from jax import lax
from jax.experimental import pallas as pl
from jax.experimental.pallas import tpu as pltpu
from jax.experimental.pallas import tpu_sc as plsc

N = 10000
E = 320000
D = 128
NSPEC = 4
NB = 8
HID = 8

BE = 6400
BN = 2000
EPB = 16


def _edge_w_body(emb_ref, ai_ref, rep_ref, wfc1bd_ref, wfc2e_ref, out_ref):
    h = jnp.dot(emb_ref[...], wfc1bd_ref[...],
                preferred_element_type=jnp.float32) * (1.0 / math.sqrt(NB))
    h = h * jax.nn.sigmoid(h)
    a_int = jnp.dot(ai_ref[...], rep_ref[...], preferred_element_type=jnp.float32)
    h = h * a_int * (1.0 / math.sqrt(HID))
    wfc2e = wfc2e_ref[...]
    parts = [jnp.dot(h, wfc2e[t], preferred_element_type=jnp.float32)
             for t in range(EPB)]
    out_ref[...] = jnp.concatenate(parts, axis=0)


def _edge_w(emb2, ai16, rep, wfc1bd, wfc2e, pe):
    R = BE // EPB
    return pl.pallas_call(
        _edge_w_body,
        grid=(pe // BE,),
        in_specs=[
            pl.BlockSpec((R, D), lambda i: (i, 0)),
            pl.BlockSpec((R, EPB), lambda i: (i, 0)),
            pl.BlockSpec((EPB, D), lambda i: (0, 0)),
            pl.BlockSpec((D, D), lambda i: (0, 0)),
            pl.BlockSpec((EPB, D, D), lambda i: (0, 0, 0)),
        ],
        out_specs=pl.BlockSpec((BE, D), lambda i: (i, 0)),
        out_shape=jax.ShapeDtypeStruct((pe, D), jnp.float32),
    )(emb2, ai16, rep, wfc1bd, wfc2e)


def _node_pre_body(nf_ref, attrs_ref, w1_ref, wsct_ref, xw_ref, sc_ref):
    nf = nf_ref[...]
    xw_ref[...] = jnp.dot(nf, w1_ref[...],
                          preferred_element_type=jnp.float32) * (1.0 / math.sqrt(D))
    wsct = wsct_ref[...]
    attrs = attrs_ref[...]
    acc = attrs[:, 0:1] * jnp.dot(nf, wsct[0], preferred_element_type=jnp.float32)
    for j in range(1, NSPEC):
        acc = acc + attrs[:, j:j + 1] * jnp.dot(
            nf, wsct[j], preferred_element_type=jnp.float32)
    sc_ref[...] = acc * (1.0 / math.sqrt(D * NSPEC))


def _node_pre(nf, attrs, w1, wsct):
    return pl.pallas_call(
        _node_pre_body,
        grid=(N // BN,),
        in_specs=[
            pl.BlockSpec((BN, D), lambda i: (i, 0)),
            pl.BlockSpec((BN, NSPEC), lambda i: (i, 0)),
            pl.BlockSpec((D, D), lambda i: (0, 0)),
            pl.BlockSpec((NSPEC, D, D), lambda i: (0, 0, 0)),
        ],
        out_specs=[
            pl.BlockSpec((BN, D), lambda i: (i, 0)),
            pl.BlockSpec((BN, D), lambda i: (i, 0)),
        ],
        out_shape=[
            jax.ShapeDtypeStruct((N, D), jnp.float32),
            jax.ShapeDtypeStruct((N, D), jnp.float32),
        ],
    )(nf, attrs, w1, wsct)


NC = 2
NS = 16
NW = NC * NS
C = 80
NPT = 624


RPB = BE // EPB
WL = C * EPB


def _sc_body(ept, nch,
             xw_hbm, w_hbm, src_hbm, dst_hbm, out_hbm,
             srcw0, srcw1, dstw0, dstw1, sidx0, sidx1, didx0, didx1,
             rows0, rows1, wbuf0, wbuf1,
             acc, ssw0, ssw1, sdw0, sdw1, sw0, sw1, sg0, sg1):
    c = lax.axis_index("c")
    s = lax.axis_index("s")
    wid = c * NS + s
    ebase = wid * ept
    nbase = s * NPT
    srcw = (srcw0, srcw1)
    dstw = (dstw0, dstw1)
    sidx = (sidx0, sidx1)
    didx = (didx0, didx1)
    rows = (rows0, rows1)
    wbuf = (wbuf0, wbuf1)
    ssw = (ssw0, ssw1)
    sdw = (sdw0, sdw1)
    sw = (sw0, sw1)
    sg = (sg0, sg1)

    def _zero(i, _):
        for j in range(D // 16):
            rows0[i, pl.ds(j * 16, 16)] = jnp.zeros((16,), jnp.float32)
        return 0

    lax.fori_loop(0, C, _zero, 0)
    for k in range(NPT // C):
        pltpu.sync_copy(rows0, acc.at[pl.ds(nbase + k * C, C)])
    rem = NPT % C
    if rem:
        pltpu.sync_copy(rows0.at[pl.ds(0, rem)],
                        acc.at[pl.ds(nbase + (NPT // C) * C, rem)])

    @pl.when(s == NS - 1)
    def _zero_tail():
        pltpu.sync_copy(rows0.at[pl.ds(0, N - NS * NPT)],
                        acc.at[pl.ds(NS * NPT, N - NS * NPT)])

    plsc.subcore_barrier()

    def _winparams(i):
        P = ebase + i * C
        g = P // BE
        q = P % BE
        t = q // RPB
        W = g * BE + EPB * (q % RPB)
        return P, W, t

    def _idx_w(i, b):
        P, W, _ = _winparams(i)
        pltpu.async_copy(src_hbm.at[pl.ds(W, WL)], srcw[b], ssw[b])
        pltpu.async_copy(dst_hbm.at[pl.ds(W, WL)], dstw[b], sdw[b])
        pltpu.async_copy(w_hbm.at[pl.ds(P, C)], wbuf[b], sw[b])

    def _extract_gather(i, b):
        _, _, t = _winparams(i)
        pltpu.make_async_copy(src_hbm.at[pl.ds(0, WL)], srcw[b], ssw[b]).wait()
        pltpu.make_async_copy(dst_hbm.at[pl.ds(0, WL)], dstw[b], sdw[b]).wait()
        iota = lax.broadcasted_iota(jnp.int32, (16,), 0) * EPB
        for c5 in range(C // 16):
            idx = iota + (16 * EPB * c5 + t)
            sidx[b][pl.ds(c5 * 16, 16)] = plsc.load_gather(srcw[b], [idx])
            didx[b][pl.ds(c5 * 16, 16)] = plsc.load_gather(dstw[b], [idx])
        pltpu.async_copy(xw_hbm.at[sidx[b]], rows[b], sg[b])

    def _step(i, b, prefetch):
        if prefetch:
            _idx_w(i + 1, b ^ 1)
        pltpu.make_async_copy(w_hbm.at[pl.ds(0, C)], wbuf[b], sw[b]).wait()
        pltpu.make_async_copy(xw_hbm.at[sidx[b]], rows[b], sg[b]).wait()

        def _mul(e, _):
            for j in range(D // 16):
                sl = pl.ds(j * 16, 16)
                rows[b][e, sl] = rows[b][e, sl] * wbuf[b][e, sl]
            return 0

        lax.fori_loop(0, C, _mul, 0)
        if prefetch:
            _extract_gather(i + 1, b ^ 1)
        pltpu.sync_copy(rows[b], acc.at[didx[b]], add=True)

    _idx_w(0, 0)
    _extract_gather(0, 0)

    def _pair(i2, _):
        i = i2 * 2
        _step(i, 0, True)
        _step(i + 1, 1, True)
        return 0

    lax.fori_loop(0, (nch - 1) // 2, _pair, 0)
    if nch % 2 == 0:
        _step(nch - 2, (nch - 2) & 1, True)
    _step(nch - 1, (nch - 1) & 1, False)

    plsc.subcore_barrier()
    pltpu.sync_copy(acc.at[pl.ds(nbase, NPT)],
                    out_hbm.at[c, pl.ds(nbase, NPT)])

    @pl.when(s == NS - 1)
    def _out_tail():
        pltpu.sync_copy(acc.at[pl.ds(NS * NPT, N - NS * NPT)],
                        out_hbm.at[c, pl.ds(NS * NPT, N - NS * NPT)])


def _sc_scatter(xw, w, src, dst, pe):
    mesh = plsc.VectorSubcoreMesh(core_axis_name="c", subcore_axis_name="s")
    ept = pe // NW
    f = pl.kernel(
        functools.partial(_sc_body, ept, ept // C),
        out_type=jax.ShapeDtypeStruct((NC, N, D), jnp.float32),
        mesh=mesh,
        compiler_params=pltpu.CompilerParams(needs_layout_passes=False),
        scratch_types=(
            [pltpu.VMEM((WL,), jnp.int32)] * 4
            + [pltpu.VMEM((C,), jnp.int32)] * 4
            + [pltpu.VMEM((C, D), jnp.float32)] * 4
            + [pltpu.VMEM_SHARED((N, D), jnp.float32)]
            + [pltpu.SemaphoreType.DMA] * 8
        ),
    )
    return f(xw, w, src, dst)


def _final_body(p_ref, q_ref, sc_ref, w2_ref, out_ref):
    ssum = p_ref[0] + p_ref[1] + q_ref[0] + q_ref[1]
    out_ref[...] = jnp.dot(ssum, w2_ref[...],
                           preferred_element_type=jnp.float32) * (1.0 / math.sqrt(D)) + sc_ref[...]


def _final(partials0, partials1, scv, w2):
    return pl.pallas_call(
        _final_body,
        grid=(N // BN,),
        in_specs=[
            pl.BlockSpec((NC, BN, D), lambda i: (0, i, 0)),
            pl.BlockSpec((NC, BN, D), lambda i: (0, i, 0)),
            pl.BlockSpec((BN, D), lambda i: (i, 0)),
            pl.BlockSpec((D, D), lambda i: (0, 0)),
        ],
        out_specs=pl.BlockSpec((BN, D), lambda i: (i, 0)),
        out_shape=jax.ShapeDtypeStruct((N, D), jnp.float32),
    )(partials0, partials1, scv, w2)


import numpy as np

_REP = np.zeros((EPB, D), np.float32)
_REP[np.arange(EPB)[:, None], np.arange(EPB * NB).reshape(EPB, NB)] = 1.0


_PHASES = (166400, 153600)


def kernel(node_features, node_attrs, edge_attrs, edge_embedding,
           W1, Wfc1, Wfc2, W2, Wsc, edge_index):
    src = edge_index[1]
    dst = edge_index[0]
    wsct = jnp.transpose(Wsc, (1, 0, 2))
    rep = jnp.asarray(_REP)
    wfc1bd = jnp.kron(jnp.eye(EPB, dtype=jnp.float32), Wfc1)
    wfc2e = jnp.zeros((EPB, D, D), jnp.float32).at[
        np.arange(EPB)[:, None], np.arange(EPB * NB).reshape(EPB, NB)].set(Wfc2)
    xw, scv = _node_pre(node_features, node_attrs, W1, wsct)
    partials = []
    off = 0
    for pe in _PHASES:
        sl = slice(off, off + pe)
        emb2 = edge_embedding[sl].reshape(pe // EPB, D)
        ai16 = edge_attrs[sl].reshape(pe // EPB, EPB)
        w = _edge_w(emb2, ai16, rep, wfc1bd, wfc2e, pe)
        partials.append(_sc_scatter(xw, w, src[sl], dst[sl], pe))
        off += pe
    return _final(partials[0], partials[1], scv, W2)


# speedup vs baseline: 1.2183x; 1.0417x over previous
import functools
import math

import jax
import jax.numpy as jnp
from jax import lax
from jax.experimental import pallas as pl
from jax.experimental.pallas import tpu as pltpu
from jax.experimental.pallas import tpu_sc as plsc

N = 10000
E = 320000
D = 128
NSPEC = 4
NB = 8
HID = 8

BE = 6400
BN = 2000
EPB = 16


def _edge_w_body(emb_ref, ai_ref, rep_ref, wfc1bd_ref, wfc2e_ref, out_ref):
    h = jnp.dot(emb_ref[...], wfc1bd_ref[...],
                preferred_element_type=jnp.float32) * (1.0 / math.sqrt(NB))
    h = h * jax.nn.sigmoid(h)
    a_int = jnp.dot(ai_ref[...], rep_ref[...], preferred_element_type=jnp.float32)
    h = h * a_int * (1.0 / math.sqrt(HID))
    wfc2e = wfc2e_ref[...]
    parts = [jnp.dot(h, wfc2e[t], preferred_element_type=jnp.float32)
             for t in range(EPB)]
    out_ref[...] = jnp.concatenate(parts, axis=0)


def _edge_w(emb2, ai16, rep, wfc1bd, wfc2e, pe):
    R = BE // EPB
    return pl.pallas_call(
        _edge_w_body,
        grid=(pe // BE,),
        in_specs=[
            pl.BlockSpec((R, D), lambda i: (i, 0)),
            pl.BlockSpec((R, EPB), lambda i: (i, 0)),
            pl.BlockSpec((EPB, D), lambda i: (0, 0)),
            pl.BlockSpec((D, D), lambda i: (0, 0)),
            pl.BlockSpec((EPB, D, D), lambda i: (0, 0, 0)),
        ],
        out_specs=pl.BlockSpec((BE, D), lambda i: (i, 0)),
        out_shape=jax.ShapeDtypeStruct((pe, D), jnp.float32),
    )(emb2, ai16, rep, wfc1bd, wfc2e)


def _node_pre_body(nf_ref, attrs_ref, w1_ref, wsct_ref, xw_ref, sc_ref):
    nf = nf_ref[...]
    xw_ref[...] = jnp.dot(nf, w1_ref[...],
                          preferred_element_type=jnp.float32) * (1.0 / math.sqrt(D))
    wsct = wsct_ref[...]
    attrs = attrs_ref[...]
    acc = attrs[:, 0:1] * jnp.dot(nf, wsct[0], preferred_element_type=jnp.float32)
    for j in range(1, NSPEC):
        acc = acc + attrs[:, j:j + 1] * jnp.dot(
            nf, wsct[j], preferred_element_type=jnp.float32)
    sc_ref[...] = acc * (1.0 / math.sqrt(D * NSPEC))


def _node_pre(nf, attrs, w1, wsct):
    return pl.pallas_call(
        _node_pre_body,
        grid=(N // BN,),
        in_specs=[
            pl.BlockSpec((BN, D), lambda i: (i, 0)),
            pl.BlockSpec((BN, NSPEC), lambda i: (i, 0)),
            pl.BlockSpec((D, D), lambda i: (0, 0)),
            pl.BlockSpec((NSPEC, D, D), lambda i: (0, 0, 0)),
        ],
        out_specs=[
            pl.BlockSpec((BN, D), lambda i: (i, 0)),
            pl.BlockSpec((BN, D), lambda i: (i, 0)),
        ],
        out_shape=[
            jax.ShapeDtypeStruct((N, D), jnp.float32),
            jax.ShapeDtypeStruct((N, D), jnp.float32),
        ],
    )(nf, attrs, w1, wsct)


NC = 2
NS = 16
NW = NC * NS
C = 80
NPT = 624


RPB = BE // EPB
WL = C * EPB


def _sc_body(ept, nch,
             xw_hbm, w_hbm, src_hbm, dst_hbm, out_hbm,
             srcw0, srcw1, dstw0, dstw1, sidx0, sidx1, didx0, didx1,
             rows0, rows1, wbuf0, wbuf1,
             acc, ssw0, ssw1, sdw0, sdw1, sw0, sw1, sg0, sg1, ssc0, ssc1):
    c = lax.axis_index("c")
    s = lax.axis_index("s")
    wid = c * NS + s
    ebase = wid * ept
    nbase = s * NPT
    srcw = (srcw0, srcw1)
    dstw = (dstw0, dstw1)
    sidx = (sidx0, sidx1)
    didx = (didx0, didx1)
    rows = (rows0, rows1)
    wbuf = (wbuf0, wbuf1)
    ssw = (ssw0, ssw1)
    sdw = (sdw0, sdw1)
    sw = (sw0, sw1)
    sg = (sg0, sg1)
    ssc = (ssc0, ssc1)

    def _zero(i, _):
        for j in range(D // 16):
            rows0[i, pl.ds(j * 16, 16)] = jnp.zeros((16,), jnp.float32)
        return 0

    lax.fori_loop(0, C, _zero, 0)
    for k in range(NPT // C):
        pltpu.sync_copy(rows0, acc.at[pl.ds(nbase + k * C, C)])
    rem = NPT % C
    if rem:
        pltpu.sync_copy(rows0.at[pl.ds(0, rem)],
                        acc.at[pl.ds(nbase + (NPT // C) * C, rem)])

    @pl.when(s == NS - 1)
    def _zero_tail():
        pltpu.sync_copy(rows0.at[pl.ds(0, N - NS * NPT)],
                        acc.at[pl.ds(NS * NPT, N - NS * NPT)])

    plsc.subcore_barrier()

    def _winparams(i):
        P = ebase + i * C
        g = P // BE
        q = P % BE
        t = q // RPB
        W = g * BE + EPB * (q % RPB)
        return P, W, t

    def _idx_w(i, b):
        P, W, _ = _winparams(i)
        pltpu.async_copy(src_hbm.at[pl.ds(W, WL)], srcw[b], ssw[b])
        pltpu.async_copy(dst_hbm.at[pl.ds(W, WL)], dstw[b], sdw[b])
        pltpu.async_copy(w_hbm.at[pl.ds(P, C)], wbuf[b], sw[b])

    def _extract_gather(i, b):
        _, _, t = _winparams(i)
        pltpu.make_async_copy(src_hbm.at[pl.ds(0, WL)], srcw[b], ssw[b]).wait()
        pltpu.make_async_copy(dst_hbm.at[pl.ds(0, WL)], dstw[b], sdw[b]).wait()
        iota = lax.broadcasted_iota(jnp.int32, (16,), 0) * EPB
        for c5 in range(C // 16):
            idx = iota + (16 * EPB * c5 + t)
            sidx[b][pl.ds(c5 * 16, 16)] = plsc.load_gather(srcw[b], [idx])
            didx[b][pl.ds(c5 * 16, 16)] = plsc.load_gather(dstw[b], [idx])
        pltpu.async_copy(xw_hbm.at[sidx[b]], rows[b], sg[b])

    def _step(i, b, prefetch, first=False):
        if prefetch:
            _idx_w(i + 1, b ^ 1)
        pltpu.make_async_copy(w_hbm.at[pl.ds(0, C)], wbuf[b], sw[b]).wait()
        pltpu.make_async_copy(xw_hbm.at[sidx[b]], rows[b], sg[b]).wait()

        @functools.partial(plsc.parallel_loop, 0, C, unroll=4)
        def _mul(e):
            for j in range(D // 16):
                sl = pl.ds(j * 16, 16)
                rows[b][e, sl] = rows[b][e, sl] * wbuf[b][e, sl]

        if prefetch:
            if not first:
                pltpu.make_async_copy(rows[b ^ 1], acc.at[didx[b ^ 1]],
                                      ssc[b ^ 1]).wait()
            _extract_gather(i + 1, b ^ 1)
        pltpu.async_copy(rows[b], acc.at[didx[b]], ssc[b], add=True)

    _idx_w(0, 0)
    _extract_gather(0, 0)
    _step(0, 0, True, first=True)

    def _pair(i2, _):
        i = i2 * 2 + 1
        _step(i, 1, True)
        _step(i + 1, 0, True)
        return 0

    lax.fori_loop(0, (nch - 2) // 2, _pair, 0)
    if nch % 2 == 1:
        _step(nch - 2, (nch - 2) & 1, True)
    _step(nch - 1, (nch - 1) & 1, False)
    pltpu.make_async_copy(rows[(nch - 2) & 1], acc.at[didx[(nch - 2) & 1]],
                          ssc[(nch - 2) & 1]).wait()
    pltpu.make_async_copy(rows[(nch - 1) & 1], acc.at[didx[(nch - 1) & 1]],
                          ssc[(nch - 1) & 1]).wait()

    plsc.subcore_barrier()
    pltpu.sync_copy(acc.at[pl.ds(nbase, NPT)],
                    out_hbm.at[c, pl.ds(nbase, NPT)])

    @pl.when(s == NS - 1)
    def _out_tail():
        pltpu.sync_copy(acc.at[pl.ds(NS * NPT, N - NS * NPT)],
                        out_hbm.at[c, pl.ds(NS * NPT, N - NS * NPT)])


def _sc_scatter(xw, w, src, dst, pe):
    mesh = plsc.VectorSubcoreMesh(core_axis_name="c", subcore_axis_name="s")
    ept = pe // NW
    f = pl.kernel(
        functools.partial(_sc_body, ept, ept // C),
        out_type=jax.ShapeDtypeStruct((NC, N, D), jnp.float32),
        mesh=mesh,
        compiler_params=pltpu.CompilerParams(needs_layout_passes=False),
        scratch_types=(
            [pltpu.VMEM((WL,), jnp.int32)] * 4
            + [pltpu.VMEM((C,), jnp.int32)] * 4
            + [pltpu.VMEM((C, D), jnp.float32)] * 4
            + [pltpu.VMEM_SHARED((N, D), jnp.float32)]
            + [pltpu.SemaphoreType.DMA] * 10
        ),
    )
    return f(xw, w, src, dst)


def _final_body(p_ref, q_ref, sc_ref, w2_ref, out_ref):
    ssum = p_ref[0] + p_ref[1] + q_ref[0] + q_ref[1]
    out_ref[...] = jnp.dot(ssum, w2_ref[...],
                           preferred_element_type=jnp.float32) * (1.0 / math.sqrt(D)) + sc_ref[...]


def _final(partials0, partials1, scv, w2):
    return pl.pallas_call(
        _final_body,
        grid=(N // BN,),
        in_specs=[
            pl.BlockSpec((NC, BN, D), lambda i: (0, i, 0)),
            pl.BlockSpec((NC, BN, D), lambda i: (0, i, 0)),
            pl.BlockSpec((BN, D), lambda i: (i, 0)),
            pl.BlockSpec((D, D), lambda i: (0, 0)),
        ],
        out_specs=pl.BlockSpec((BN, D), lambda i: (i, 0)),
        out_shape=jax.ShapeDtypeStruct((N, D), jnp.float32),
    )(partials0, partials1, scv, w2)


import numpy as np

_REP = np.zeros((EPB, D), np.float32)
_REP[np.arange(EPB)[:, None], np.arange(EPB * NB).reshape(EPB, NB)] = 1.0


_PHASES = (166400, 153600)


def kernel(node_features, node_attrs, edge_attrs, edge_embedding,
           W1, Wfc1, Wfc2, W2, Wsc, edge_index):
    src = edge_index[1]
    dst = edge_index[0]
    wsct = jnp.transpose(Wsc, (1, 0, 2))
    rep = jnp.asarray(_REP)
    wfc1bd = jnp.kron(jnp.eye(EPB, dtype=jnp.float32), Wfc1)
    wfc2e = jnp.zeros((EPB, D, D), jnp.float32).at[
        np.arange(EPB)[:, None], np.arange(EPB * NB).reshape(EPB, NB)].set(Wfc2)
    xw, scv = _node_pre(node_features, node_attrs, W1, wsct)
    partials = []
    off = 0
    for pe in _PHASES:
        sl = slice(off, off + pe)
        emb2 = edge_embedding[sl].reshape(pe // EPB, D)
        ai16 = edge_attrs[sl].reshape(pe // EPB, EPB)
        w = _edge_w(emb2, ai16, rep, wfc1bd, wfc2e, pe)
        partials.append(_sc_scatter(xw, w, src[sl], dst[sl], pe))
        off += pe
    return _final(partials[0], partials[1], scv, W2)
